# Initial kernel scaffold; baseline (speedup 1.0000x reference)
import jax
import jax.numpy as jnp
from jax.experimental import pallas as pl


def kernel(mel_spec, w_taps, bias_col):
    raise NotImplementedError("write your pallas kernel here")


# speedup vs baseline: 1.0235x; 1.0235x over previous
import functools

import jax
import jax.numpy as jnp
from jax.experimental import pallas as pl
from jax.experimental.pallas import tpu as pltpu


def _enhancer_kernel(w_ref, b_ref, x_ref, o_ref, *, T):
    x = x_ref[...]
    xb = x.astype(jnp.bfloat16)

    t = jax.lax.broadcasted_iota(jnp.int32, (1, T), 1)
    zero = jnp.bfloat16(0)
    x_prev = jnp.where(t == 0, zero, pltpu.roll(xb, shift=1, axis=1))
    x_next = jnp.where(t == T - 1, zero, pltpu.roll(xb, shift=T - 1, axis=1))

    y = jnp.dot(w_ref[0], x_prev, preferred_element_type=jnp.float32)
    y = y + jnp.dot(w_ref[1], xb, preferred_element_type=jnp.float32)
    y = y + jnp.dot(w_ref[2], x_next, preferred_element_type=jnp.float32)

    o_ref[...] = y + b_ref[...] + jnp.float32(0.7) * x


def kernel(mel_spec, w_taps, bias_col):
    B, M, T = mel_spec.shape
    w_bf16 = w_taps.astype(jnp.bfloat16)

    return pl.pallas_call(
        functools.partial(_enhancer_kernel, T=T),
        out_shape=jax.ShapeDtypeStruct((B, M, T), mel_spec.dtype),
        grid=(B,),
        in_specs=[
            pl.BlockSpec((3, M, M), lambda b: (0, 0, 0)),
            pl.BlockSpec((M, 1), lambda b: (0, 0)),
            pl.BlockSpec((None, M, T), lambda b: (b, 0, 0)),
        ],
        out_specs=pl.BlockSpec((None, M, T), lambda b: (b, 0, 0)),
        compiler_params=pltpu.CompilerParams(
            dimension_semantics=("parallel",),
            vmem_limit_bytes=48 << 20,
        ),
    )(w_bf16, bias_col, mel_spec)


# speedup vs baseline: 1.5666x; 1.5307x over previous
import functools

import jax
import jax.numpy as jnp
from jax.experimental import pallas as pl
from jax.experimental.pallas import tpu as pltpu


def _enhancer_kernel(w_ref, b_ref, x_ref, o_ref, *, T, BB):
    t = jax.lax.broadcasted_iota(jnp.int32, (1, T), 1)
    m_first = t == 0
    m_last = t == T - 1
    w0, w1, w2 = w_ref[0], w_ref[1], w_ref[2]
    bias = b_ref[...]
    zero = jnp.bfloat16(0)
    for i in range(BB):
        x = x_ref[i]
        xb = x.astype(jnp.bfloat16)
        x_prev = jnp.where(m_first, zero, pltpu.roll(xb, shift=1, axis=1))
        x_next = jnp.where(m_last, zero, pltpu.roll(xb, shift=T - 1, axis=1))
        y = jnp.dot(w0, x_prev, preferred_element_type=jnp.float32)
        y = y + jnp.dot(w1, xb, preferred_element_type=jnp.float32)
        y = y + jnp.dot(w2, x_next, preferred_element_type=jnp.float32)
        o_ref[i] = y + bias + jnp.float32(0.7) * x


def kernel(mel_spec, w_taps, bias_col):
    B, M, T = mel_spec.shape
    BB = 8
    w_bf16 = w_taps.astype(jnp.bfloat16)

    return pl.pallas_call(
        functools.partial(_enhancer_kernel, T=T, BB=BB),
        out_shape=jax.ShapeDtypeStruct((B, M, T), mel_spec.dtype),
        grid=(B // BB,),
        in_specs=[
            pl.BlockSpec((3, M, M), lambda b: (0, 0, 0)),
            pl.BlockSpec((M, 1), lambda b: (0, 0)),
            pl.BlockSpec((BB, M, T), lambda b: (b, 0, 0)),
        ],
        out_specs=pl.BlockSpec((BB, M, T), lambda b: (b, 0, 0)),
        compiler_params=pltpu.CompilerParams(
            dimension_semantics=("parallel",),
            vmem_limit_bytes=64 << 20,
        ),
    )(w_bf16, bias_col, mel_spec)


# speedup vs baseline: 1.6156x; 1.0313x over previous
import functools

import jax
import jax.numpy as jnp
from jax.experimental import pallas as pl
from jax.experimental.pallas import tpu as pltpu


def _enhancer_kernel(w_ref, b_ref, x_ref, o_ref, *, T, BB):
    t = jax.lax.broadcasted_iota(jnp.int32, (1, T), 1)
    m_first = t == 0
    m_last = t == T - 1
    w0, w1, w2 = w_ref[0], w_ref[1], w_ref[2]
    bias = b_ref[...]
    zero = jnp.bfloat16(0)
    for i in range(BB):
        xb = x_ref[i].astype(jnp.bfloat16)
        x_prev = jnp.where(m_first, zero, pltpu.roll(xb, shift=1, axis=1))
        x_next = jnp.where(m_last, zero, pltpu.roll(xb, shift=T - 1, axis=1))
        y = jnp.dot(w0, x_prev, preferred_element_type=jnp.float32)
        y = y + jnp.dot(w1, xb, preferred_element_type=jnp.float32)
        y = y + jnp.dot(w2, x_next, preferred_element_type=jnp.float32)
        o_ref[i] = y + bias


def kernel(mel_spec, w_taps, bias_col):
    B, M, T = mel_spec.shape
    BB = 8
    w_folded = w_taps.at[1].add(jnp.float32(0.7) * jnp.eye(M, dtype=w_taps.dtype))
    w_bf16 = w_folded.astype(jnp.bfloat16)

    return pl.pallas_call(
        functools.partial(_enhancer_kernel, T=T, BB=BB),
        out_shape=jax.ShapeDtypeStruct((B, M, T), mel_spec.dtype),
        grid=(B // BB,),
        in_specs=[
            pl.BlockSpec((3, M, M), lambda b: (0, 0, 0)),
            pl.BlockSpec((M, 1), lambda b: (0, 0)),
            pl.BlockSpec((BB, M, T), lambda b: (b, 0, 0)),
        ],
        out_specs=pl.BlockSpec((BB, M, T), lambda b: (b, 0, 0)),
        compiler_params=pltpu.CompilerParams(
            dimension_semantics=("parallel",),
            vmem_limit_bytes=64 << 20,
        ),
    )(w_bf16, bias_col, mel_spec)
